# baseline (speedup 1.0000x reference)
import functools

import jax
import jax.numpy as jnp
from jax import lax
from jax.experimental import pallas as pl
from jax.experimental.pallas import tpu as pltpu
from jax.experimental.pallas import tpu_sc as plsc

BATCH = 4096
NUM_STAGES = 10
NLAYER = NUM_STAGES + 1
CODE_LEN = 1024
CLIP = 20.0
LANES = 16
NCHUNK = CODE_LEN // LANES
BLK = 16
SIGN = jnp.int32(-2**31)
MAG = jnp.int32(2**31 - 1)


def _f2i(x):
    return lax.bitcast_convert_type(x, jnp.int32)


def _i2f(x):
    return lax.bitcast_convert_type(x, jnp.float32)


def _minsum(x, y):
    xb = _f2i(x)
    yb = _f2i(y)
    s = (xb ^ yb) & SIGN
    m = _f2i(jnp.minimum(_i2f(xb & MAG), _i2f(yb & MAG)))
    return _i2f(m | s)


def _clip(x):
    return jnp.minimum(jnp.maximum(x, -CLIP), CLIP)


def _make_sc_kernel():
    info = plsc.get_sparse_core_info()
    nc, ns = info.num_cores, info.num_subcores
    nw = nc * ns
    rows_per_w = BATCH // nw
    nblk = rows_per_w // BLK

    mesh = plsc.VectorSubcoreMesh(core_axis_name="c", subcore_axis_name="s")

    @functools.partial(
        pl.kernel,
        out_type=jax.ShapeDtypeStruct((NLAYER, BATCH, CODE_LEN), jnp.float32),
        mesh=mesh,
        scratch_types=[
            pltpu.VMEM((2, BLK, CODE_LEN), jnp.float32),
            pltpu.VMEM((2, BLK, CODE_LEN), jnp.float32),
            pltpu.VMEM((2, BLK, CODE_LEN), jnp.float32),
            pltpu.VMEM((2 * NUM_STAGES * LANES,), jnp.float32),
            pltpu.SemaphoreType.DMA,
            pltpu.SemaphoreType.DMA,
            pltpu.SemaphoreType.DMA,
            pltpu.SemaphoreType.DMA,
            pltpu.SemaphoreType.DMA,
            pltpu.SemaphoreType.DMA,
        ],
    )
    def sc_kernel(right_hbm, left_hbm, w_hbm, out_hbm,
                  lbuf, rin, oout, wbuf,
                  lsem0, lsem1, rsem0, rsem1, osem0, osem1):
        wid = lax.axis_index("s") * nc + lax.axis_index("c")
        base = wid * rows_per_w
        lsem = (lsem0, lsem1)
        rsem = (rsem0, rsem1)
        osem = (osem0, osem1)
        pltpu.sync_copy(w_hbm, wbuf)
        lanes = lax.iota(jnp.int32, 16)
        w0s = [wbuf[pl.ds(i * LANES, LANES)] for i in range(NUM_STAGES)]
        w1s = [wbuf[pl.ds((NUM_STAGES + i) * LANES, LANES)]
               for i in range(NUM_STAGES)]

        def issue_l(slot, b):
            pltpu.async_copy(left_hbm.at[NUM_STAGES, pl.ds(base + b * BLK, BLK)],
                             lbuf.at[slot], lsem[slot])

        def wait_l(slot, b):
            pltpu.make_async_copy(
                left_hbm.at[NUM_STAGES, pl.ds(base + b * BLK, BLK)],
                lbuf.at[slot], lsem[slot]).wait()

        def issue_r(slot, b, layer):
            pltpu.async_copy(right_hbm.at[layer, pl.ds(base + b * BLK, BLK)],
                             rin.at[slot], rsem[slot])

        def wait_r(slot, b, layer):
            pltpu.make_async_copy(
                right_hbm.at[layer, pl.ds(base + b * BLK, BLK)],
                rin.at[slot], rsem[slot]).wait()

        def issue_o(slot, b, layer):
            pltpu.async_copy(oout.at[slot],
                             out_hbm.at[layer, pl.ds(base + b * BLK, BLK)],
                             osem[slot])

        def wait_o(slot, b, layer):
            pltpu.make_async_copy(
                oout.at[slot],
                out_hbm.at[layer, pl.ds(base + b * BLK, BLK)],
                osem[slot]).wait()

        issue_l(0, 0)
        issue_l(1, 1)
        issue_r(0, 0, 9)
        issue_r(1, 0, 8)

        def do_block(sb, b):
            wait_l(sb, b)

            oslot = sb

            @pl.when(b >= 1)
            def _():
                wait_o(oslot, b, NUM_STAGES)

            @plsc.parallel_loop(0, BLK * NCHUNK, unroll=4)
            def clip_body(v):
                row = v >> 6
                bb = (v & (NCHUNK - 1)) * LANES
                oout[oslot, row, pl.ds(bb, LANES)] = _clip(
                    lbuf[sb, row, pl.ds(bb, LANES)])
            issue_o(oslot, b, NUM_STAGES)

            for i in reversed(range(NUM_STAGES)):
                w0 = w0s[i]
                w1 = w1s[i]
                rslot = (9 - i) & 1
                oslot = (sb + NUM_STAGES - i) & 1
                wait_r(rslot, b, i)

                if i == 9:
                    @pl.when(b >= 1)
                    def _():
                        wait_o(oslot, b, i)
                else:
                    wait_o(oslot, b, i)

                if i >= 4:
                    sh = i - 4
                    db = 1 << sh

                    @plsc.parallel_loop(0, BLK * (NCHUNK // 2), unroll=8)
                    def pair_body(v, i=i, sh=sh, db=db, w0=w0, w1=w1,
                                  rslot=rslot, oslot=oslot):
                            row = v >> 5
                            t = v & (NCHUNK // 2 - 1)
                            mchunk = ((t >> sh) << (sh + 1)) | (t & (db - 1))
                            mb = mchunk * LANES
                            pb = mb + db * LANES
                            cm = lbuf[sb, row, pl.ds(mb, LANES)]
                            cp = lbuf[sb, row, pl.ds(pb, LANES)]
                            rm = rin[rslot, row, pl.ds(mb, LANES)]
                            rp = rin[rslot, row, pl.ds(pb, LANES)]
                            nm = w0 * _minsum(cm, cp + rp)
                            npv = w1 * _minsum(cm, rm) + cp
                            lbuf[sb, row, pl.ds(mb, LANES)] = nm
                            lbuf[sb, row, pl.ds(pb, LANES)] = npv
                            oout[oslot, row, pl.ds(mb, LANES)] = _clip(nm)
                            oout[oslot, row, pl.ds(pb, LANES)] = _clip(npv)
                else:
                    d = 1 << i
                    xidx = lanes ^ d
                    upper = (lanes & d) == 0

                    keep = i > 0

                    @plsc.parallel_loop(0, BLK * NCHUNK // 2, unroll=2)
                    def chunk_body(v, i=i, xidx=xidx, upper=upper,
                                   w0=w0, w1=w1, rslot=rslot, oslot=oslot,
                                   keep=keep):
                            row = v >> 5
                            b0 = (v & (NCHUNK // 2 - 1)) * (2 * LANES)
                            for bb in (b0, b0 + LANES):
                                cc = lbuf[sb, row, pl.ds(bb, LANES)]
                                rc = rin[rslot, row, pl.ds(bb, LANES)]
                                cs = cc.at[xidx].get(mode="promise_in_bounds")
                                rs = rc.at[xidx].get(mode="promise_in_bounds")
                                up = w0 * _minsum(cc, cs + rs)
                                lo = w1 * _minsum(cs, rs) + cc
                                nv = jnp.where(upper, up, lo)
                                if keep:
                                    lbuf[sb, row, pl.ds(bb, LANES)] = nv
                                oout[oslot, row, pl.ds(bb, LANES)] = _clip(nv)

                issue_o(oslot, b, i)

                if i >= 2:
                    issue_r(rslot, b, i - 2)
                else:
                    nxt_layer = 9 if i == 1 else 8

                    @pl.when(b + 1 < nblk)
                    def _():
                        issue_r(rslot, b + 1, nxt_layer)

            @pl.when(b + 2 < nblk)
            def _():
                issue_l(sb, b + 2)

        def block_pair(j, carry):
            do_block(0, 2 * j)
            do_block(1, 2 * j + 1)
            return carry

        lax.fori_loop(0, nblk // 2, block_pair, 0)

        lastb = nblk - 1
        sb = lastb & 1
        wait_o((sb + 9) & 1, lastb, 1)
        wait_o((sb + 10) & 1, lastb, 0)

    return sc_kernel


_SC_KERNEL = None


def kernel(right, left, iter, left_weights):
    global _SC_KERNEL
    if _SC_KERNEL is None:
        _SC_KERNEL = _make_sc_kernel()
    w = left_weights[iter]
    wv = jnp.broadcast_to(
        w.T[:, :, None], (2, NUM_STAGES, LANES)
    ).astype(jnp.float32).reshape(2 * NUM_STAGES * LANES)
    right_t = jnp.transpose(right, (1, 0, 2))
    left_t = jnp.transpose(left, (1, 0, 2))
    out_t = _SC_KERNEL(right_t, left_t, wv)
    return jnp.transpose(out_t, (1, 0, 2))


# speedup vs baseline: 1.0061x; 1.0061x over previous
import functools

import jax
import jax.numpy as jnp
from jax import lax
from jax.experimental import pallas as pl
from jax.experimental.pallas import tpu as pltpu
from jax.experimental.pallas import tpu_sc as plsc

BATCH = 4096
NUM_STAGES = 10
NLAYER = NUM_STAGES + 1
CODE_LEN = 1024
CLIP = 20.0
LANES = 16
NCHUNK = CODE_LEN // LANES
BLK = 16
SIGN = jnp.int32(-2**31)
MAG = jnp.int32(2**31 - 1)


def _f2i(x):
    return lax.bitcast_convert_type(x, jnp.int32)


def _i2f(x):
    return lax.bitcast_convert_type(x, jnp.float32)


def _minsum(x, y):
    xb = _f2i(x)
    yb = _f2i(y)
    s = (xb ^ yb) & SIGN
    m = _f2i(jnp.minimum(_i2f(xb & MAG), _i2f(yb & MAG)))
    return _i2f(m | s)


def _clip(x):
    return jnp.minimum(jnp.maximum(x, -CLIP), CLIP)


def _make_sc_kernel():
    info = plsc.get_sparse_core_info()
    nc, ns = info.num_cores, info.num_subcores
    nw = nc * ns
    rows_per_w = BATCH // nw
    nblk = rows_per_w // BLK

    mesh = plsc.VectorSubcoreMesh(core_axis_name="c", subcore_axis_name="s")

    @functools.partial(
        pl.kernel,
        out_type=jax.ShapeDtypeStruct((NLAYER, BATCH, CODE_LEN), jnp.float32),
        mesh=mesh,
        scratch_types=[
            pltpu.VMEM((2, BLK, CODE_LEN), jnp.float32),
            pltpu.VMEM((2, BLK, CODE_LEN), jnp.float32),
            pltpu.VMEM((2, BLK, CODE_LEN), jnp.float32),
            pltpu.VMEM((2 * NUM_STAGES * LANES,), jnp.float32),
            pltpu.SemaphoreType.DMA,
            pltpu.SemaphoreType.DMA,
            pltpu.SemaphoreType.DMA,
            pltpu.SemaphoreType.DMA,
            pltpu.SemaphoreType.DMA,
            pltpu.SemaphoreType.DMA,
        ],
    )
    def sc_kernel(right_hbm, left_hbm, w_hbm, out_hbm,
                  lbuf, rin, oout, wbuf,
                  lsem0, lsem1, rsem0, rsem1, osem0, osem1):
        wid = lax.axis_index("s") * nc + lax.axis_index("c")
        base = wid * rows_per_w
        lsem = (lsem0, lsem1)
        rsem = (rsem0, rsem1)
        osem = (osem0, osem1)
        pltpu.sync_copy(w_hbm, wbuf)
        lanes = lax.iota(jnp.int32, 16)
        w0s = [wbuf[pl.ds(i * LANES, LANES)] for i in range(NUM_STAGES)]
        w1s = [wbuf[pl.ds((NUM_STAGES + i) * LANES, LANES)]
               for i in range(NUM_STAGES)]

        def issue_l(slot, b):
            pltpu.async_copy(left_hbm.at[NUM_STAGES, pl.ds(base + b * BLK, BLK)],
                             lbuf.at[slot], lsem[slot])

        def wait_l(slot, b):
            pltpu.make_async_copy(
                left_hbm.at[NUM_STAGES, pl.ds(base + b * BLK, BLK)],
                lbuf.at[slot], lsem[slot]).wait()

        def issue_r(slot, b, layer):
            pltpu.async_copy(right_hbm.at[layer, pl.ds(base + b * BLK, BLK)],
                             rin.at[slot], rsem[slot])

        def wait_r(slot, b, layer):
            pltpu.make_async_copy(
                right_hbm.at[layer, pl.ds(base + b * BLK, BLK)],
                rin.at[slot], rsem[slot]).wait()

        def issue_o(slot, b, layer):
            pltpu.async_copy(oout.at[slot],
                             out_hbm.at[layer, pl.ds(base + b * BLK, BLK)],
                             osem[slot])

        def wait_o(slot, b, layer):
            pltpu.make_async_copy(
                oout.at[slot],
                out_hbm.at[layer, pl.ds(base + b * BLK, BLK)],
                osem[slot]).wait()

        issue_l(0, 0)
        issue_l(1, 1)
        issue_r(0, 0, 9)
        issue_r(1, 0, 8)

        def do_block(sb, b):
            wait_l(sb, b)

            oslot = sb

            @pl.when(b >= 1)
            def _():
                wait_o(oslot, b, NUM_STAGES)

            @plsc.parallel_loop(0, BLK * NCHUNK, unroll=4)
            def clip_body(v):
                row = v >> 6
                bb = (v & (NCHUNK - 1)) * LANES
                oout[oslot, row, pl.ds(bb, LANES)] = _clip(
                    lbuf[sb, row, pl.ds(bb, LANES)])
            issue_o(oslot, b, NUM_STAGES)

            for i in reversed(range(NUM_STAGES)):
                w0 = w0s[i]
                w1 = w1s[i]
                rslot = (9 - i) & 1
                oslot = (sb + NUM_STAGES - i) & 1
                wait_r(rslot, b, i)

                if i == 9:
                    @pl.when(b >= 1)
                    def _():
                        wait_o(oslot, b, i)
                else:
                    wait_o(oslot, b, i)

                if i >= 4:
                    sh = i - 4
                    db = 1 << sh

                    @plsc.parallel_loop(0, BLK * (NCHUNK // 2), unroll=8)
                    def pair_body(v, i=i, sh=sh, db=db, w0=w0, w1=w1,
                                  rslot=rslot, oslot=oslot):
                            row = v >> 5
                            t = v & (NCHUNK // 2 - 1)
                            mchunk = ((t >> sh) << (sh + 1)) | (t & (db - 1))
                            mb = mchunk * LANES
                            pb = mb + db * LANES
                            cm = lbuf[sb, row, pl.ds(mb, LANES)]
                            cp = lbuf[sb, row, pl.ds(pb, LANES)]
                            rm = rin[rslot, row, pl.ds(mb, LANES)]
                            rp = rin[rslot, row, pl.ds(pb, LANES)]
                            nm = w0 * _minsum(cm, cp + rp)
                            npv = w1 * _minsum(cm, rm) + cp
                            lbuf[sb, row, pl.ds(mb, LANES)] = nm
                            lbuf[sb, row, pl.ds(pb, LANES)] = npv
                            oout[oslot, row, pl.ds(mb, LANES)] = _clip(nm)
                            oout[oslot, row, pl.ds(pb, LANES)] = _clip(npv)
                else:
                    d = 1 << i
                    xidx = lanes ^ d
                    upper = (lanes & d) == 0

                    keep = i > 0

                    @plsc.parallel_loop(0, BLK * NCHUNK, unroll=2)
                    def chunk_body(v, i=i, xidx=xidx, upper=upper,
                                   w0=w0, w1=w1, rslot=rslot, oslot=oslot,
                                   keep=keep):
                            row = v >> 6
                            bb = (v & (NCHUNK - 1)) * LANES
                            cc = lbuf[sb, row, pl.ds(bb, LANES)]
                            rc = rin[rslot, row, pl.ds(bb, LANES)]
                            cs = cc.at[xidx].get(mode="promise_in_bounds")
                            rs = rc.at[xidx].get(mode="promise_in_bounds")
                            up = w0 * _minsum(cc, cs + rs)
                            lo = w1 * _minsum(cs, rs) + cc
                            nv = jnp.where(upper, up, lo)
                            if keep:
                                lbuf[sb, row, pl.ds(bb, LANES)] = nv
                            oout[oslot, row, pl.ds(bb, LANES)] = _clip(nv)

                issue_o(oslot, b, i)

                if i >= 2:
                    issue_r(rslot, b, i - 2)
                else:
                    nxt_layer = 9 if i == 1 else 8

                    @pl.when(b + 1 < nblk)
                    def _():
                        issue_r(rslot, b + 1, nxt_layer)

            @pl.when(b + 2 < nblk)
            def _():
                issue_l(sb, b + 2)

        def block_pair(j, carry):
            do_block(0, 2 * j)
            do_block(1, 2 * j + 1)
            return carry

        lax.fori_loop(0, nblk // 2, block_pair, 0)

        lastb = nblk - 1
        sb = lastb & 1
        wait_o((sb + 9) & 1, lastb, 1)
        wait_o((sb + 10) & 1, lastb, 0)

    return sc_kernel


_SC_KERNEL = None


def kernel(right, left, iter, left_weights):
    global _SC_KERNEL
    if _SC_KERNEL is None:
        _SC_KERNEL = _make_sc_kernel()
    w = left_weights[iter]
    wv = jnp.broadcast_to(
        w.T[:, :, None], (2, NUM_STAGES, LANES)
    ).astype(jnp.float32).reshape(2 * NUM_STAGES * LANES)
    right_t = jnp.transpose(right, (1, 0, 2))
    left_t = jnp.transpose(left, (1, 0, 2))
    out_t = _SC_KERNEL(right_t, left_t, wv)
    return jnp.transpose(out_t, (1, 0, 2))
